# baseline (speedup 1.0000x reference)
import jax
import jax.numpy as jnp
from jax.experimental import pallas as pl
from jax.experimental.pallas import tpu as pltpu

N_C, N_V, N_O = 10000, 10000, 32
HID, EMB = 64, 128


def _mlp2(x, W1, b1, W2, b2):
    return jnp.maximum(x @ W1 + b1, 0.0) @ W2 + b2


def _bn(x, g, b):
    mu = jnp.mean(x, axis=0, keepdims=True)
    var = jnp.var(x, axis=0, keepdims=True)
    return (x - mu) / jnp.sqrt(var + 1e-5) * g + b


def _encoder(x, p, n):
    h = x @ p[n + '_W1'] + p[n + '_b1']
    h = jnp.maximum(_bn(h, p[n + '_g'], p[n + '_be']), 0.0)
    return h @ p[n + '_W2'] + p[n + '_b2']


def _conv(x_src, ei, ea, n_dst, p, n):
    src, dst = ei[0], ei[1]
    msg = jnp.take(x_src, src, axis=0) * ea
    agg = jax.ops.segment_sum(msg, dst, num_segments=n_dst)
    deg = jax.ops.segment_sum(jnp.ones((ei.shape[1],), msg.dtype), dst, num_segments=n_dst)
    agg = agg / jnp.clip(deg, 1.0)[:, None]
    return _mlp2(agg, p[n + '_W1'], p[n + '_b1'], p[n + '_W2'], p[n + '_b2'])


def _pred_body(x_ref, w1_ref, b1_ref, w2_ref, b2_ref, o_ref):
    h = jnp.maximum(x_ref[...] @ w1_ref[...] + b1_ref[...], 0.0)
    o_ref[...] = (h @ w2_ref[...] + b2_ref[...])


def _pred(x, W1, b1, W2, b2):
    out = pl.pallas_call(
        _pred_body,
        out_shape=jax.ShapeDtypeStruct((x.shape[0], 1), jnp.float32),
    )(x, W1, b1[None, :], W2, b2[None, :])
    return out[:, 0]


def kernel(x_cons, x_vals, x_obj, ea_c2v, ea_v2c, ea_v2o, ea_o2v, ea_c2o, ea_o2c, params, ei_c2v, ei_v2c, ei_v2o, ei_o2v, ei_c2o, ei_o2c):
    p = params
    ei = {'c2v': ei_c2v, 'v2c': ei_v2c, 'v2o': ei_v2o, 'o2v': ei_o2v, 'c2o': ei_c2o, 'o2c': ei_o2c}
    ea = {'c2v': ea_c2v, 'v2c': ea_v2c, 'v2o': ea_v2o, 'o2v': ea_o2v, 'c2o': ea_c2o, 'o2c': ea_o2c}
    h = {'cons': _encoder(x_cons, p, 'enc_cons'),
         'vals': _encoder(x_vals, p, 'enc_vals'),
         'obj': _encoder(x_obj, p, 'enc_obj')}
    last_vals = None
    for _ in range(2):
        h2v = jnp.concatenate([_conv(h['cons'], ei['c2v'], ea['c2v'], N_V, p, 'c2v'),
                               _conv(h['obj'], ei['o2v'], ea['o2v'], N_V, p, 'o2v')], axis=1)
        h2c = jnp.concatenate([_conv(h['vals'], ei['v2c'], ea['v2c'], N_C, p, 'v2c'),
                               _conv(h['obj'], ei['o2c'], ea['o2c'], N_C, p, 'o2c')], axis=1)
        h2o = jnp.concatenate([_conv(h['vals'], ei['v2o'], ea['v2o'], N_O, p, 'v2o'),
                               _conv(h['cons'], ei['c2o'], ea['c2o'], N_O, p, 'c2o')], axis=1)
        last_vals = h2v
        h = {'vals': (jnp.maximum(h2v, 0.0) + h['vals']) * 0.5,
             'cons': (jnp.maximum(h2c, 0.0) + h['cons']) * 0.5,
             'obj': (jnp.maximum(h2o, 0.0) + h['obj']) * 0.5}
    out = _pred(last_vals, p['pred_W1'], p['pred_b1'], p['pred_W2'], p['pred_b2'])
    return out


# speedup vs baseline: 7.8351x; 7.8351x over previous
import functools

import jax
import jax.numpy as jnp
from jax import lax
from jax.experimental import pallas as pl
from jax.experimental.pallas import tpu as pltpu
from jax.experimental.pallas import tpu_sc as plsc

N_C, N_V, N_O = 10000, 10000, 32
HID, EMB = 64, 128
NUM_SC, NUM_TILES = 2, 16
NW = NUM_SC * NUM_TILES

_MESH = plsc.VectorSubcoreMesh(
    core_axis_name="c", subcore_axis_name="s", num_cores=NUM_SC,
    num_subcores=NUM_TILES)


def _make_spmm(n_src, np_dst, e_pad, K, with_deg):
    EPT = e_pad // NW
    nchunk = EPT // K
    rpt = np_dst // NUM_TILES

    out_type = [jax.ShapeDtypeStruct((NUM_SC, np_dst, HID), jnp.float32)]
    if with_deg:
        out_type.append(jax.ShapeDtypeStruct((NUM_SC, np_dst, 8),
                                             jnp.float32))

    scratch = dict(
        srcv=pltpu.VMEM((K,), jnp.int32),
        dstv=pltpu.VMEM((K,), jnp.int32),
        eav=pltpu.VMEM((K,), jnp.float32),
        rows=pltpu.VMEM((K, HID), jnp.float32),
        sagg=pltpu.VMEM_SHARED((np_dst, HID), jnp.float32),
        sem=pltpu.SemaphoreType.DMA,
    )
    if with_deg:
        scratch["onesb"] = pltpu.VMEM((K, 8), jnp.float32)
        scratch["sdeg"] = pltpu.VMEM_SHARED((np_dst, 8), jnp.float32)

    def body(y_h, src_h, dst_h, ea_h, z64_h, z8_h, o8_h, *outs, srcv, dstv,
             eav, rows, sagg, sem, onesb=None, sdeg=None):
        if with_deg:
            agg_h, deg_h = outs
        else:
            agg_h, = outs
        cid = lax.axis_index("c")
        sid = lax.axis_index("s")
        wid = cid * NUM_TILES + sid
        r0 = sid * rpt
        pltpu.sync_copy(z64_h.at[pl.ds(r0, rpt)], sagg.at[pl.ds(r0, rpt)])
        if with_deg:
            pltpu.sync_copy(z8_h.at[pl.ds(r0, rpt)], sdeg.at[pl.ds(r0, rpt)])
            pltpu.sync_copy(o8_h, onesb)
        plsc.subcore_barrier()

        base0 = wid * EPT

        def chunk(ci, carry):
            b = pl.multiple_of(base0 + ci * K, 8)
            pltpu.sync_copy(src_h.at[pl.ds(b, K)], srcv)
            pltpu.sync_copy(dst_h.at[pl.ds(b, K)], dstv)
            pltpu.sync_copy(ea_h.at[pl.ds(b, K)], eav)
            pltpu.async_copy(y_h.at[srcv], rows, sem).wait()

            def scale(g, c2):
                ea16 = eav[pl.ds(g * 16, 16)]
                for i in range(16):
                    w = ea16[i]
                    k = g * 16 + i
                    for j in range(HID // 16):
                        sl = (k, pl.ds(j * 16, 16))
                        rows[sl] = rows[sl] * w
                return c2

            lax.fori_loop(0, K // 16, scale, 0)
            pltpu.sync_copy(rows, sagg.at[dstv], add=True)
            if with_deg:
                pltpu.sync_copy(onesb, sdeg.at[dstv], add=True)
            return carry

        lax.fori_loop(0, nchunk, chunk, 0)
        plsc.subcore_barrier()
        pltpu.sync_copy(sagg.at[pl.ds(r0, rpt)], agg_h.at[cid, pl.ds(r0, rpt)])
        if with_deg:
            pltpu.sync_copy(sdeg.at[pl.ds(r0, rpt)],
                            deg_h.at[cid, pl.ds(r0, rpt)])

    return pl.kernel(
        body, out_type=out_type, mesh=_MESH, scratch_types=scratch,
        compiler_params=pltpu.CompilerParams(use_tc_tiling_on_sc=False))


@functools.lru_cache(maxsize=None)
def _spmm_fns(n_src, np_dst, e_pad, K, with_deg):
    return _make_spmm(n_src, np_dst, e_pad, K, with_deg)


def _spmm(y, src, dst, ea, n_dst, with_deg):
    e_pad = src.shape[0]
    ept = e_pad // NW
    K = 400 if ept % 400 == 0 else ept
    np_dst = ((n_dst + 127) // 128) * 128
    fn = _spmm_fns(y.shape[0], np_dst, e_pad, K, with_deg)
    z64 = jnp.zeros((np_dst, HID), jnp.float32)
    z8 = jnp.zeros((np_dst, 8), jnp.float32)
    o8 = jnp.ones((K, 8), jnp.float32)
    return fn(y, src, dst, ea, z64, z8, o8)


def _enc_body(x_ref, w1_ref, b1_ref, g_ref, be_ref, w2_ref, b2_ref, o_ref):
    h = x_ref[...] @ w1_ref[...] + b1_ref[...]
    mu = jnp.mean(h, axis=0, keepdims=True)
    var = jnp.mean((h - mu) ** 2, axis=0, keepdims=True)
    h = (h - mu) / jnp.sqrt(var + 1e-5) * g_ref[...] + be_ref[...]
    o_ref[...] = jnp.maximum(h, 0.0) @ w2_ref[...] + b2_ref[...]


def _encoder(x, p, n):
    return pl.pallas_call(
        _enc_body,
        out_shape=jax.ShapeDtypeStruct((x.shape[0], EMB), jnp.float32),
    )(x, p[n + "_W1"], p[n + "_b1"][None, :], p[n + "_g"][None, :],
      p[n + "_be"][None, :], p[n + "_W2"], p[n + "_b2"][None, :])


def _pre_body(h_ref, wa_ref, wb_ref, oa_ref, ob_ref):
    h = h_ref[...]
    oa_ref[...] = h @ wa_ref[...]
    ob_ref[...] = h @ wb_ref[...]


def _pre(h, wa, wb):
    n = h.shape[0]
    return pl.pallas_call(
        _pre_body,
        out_shape=[jax.ShapeDtypeStruct((n, HID), jnp.float32),
                   jax.ShapeDtypeStruct((n, HID), jnp.float32)],
    )(h, wa, wb)


def _post_body(aggA_ref, degA_ref, w2A_ref, b1A_ref, b2A_ref,
               aggB_ref, degB_ref, w2B_ref, b1B_ref, b2B_ref,
               hprev_ref, h2_ref, hnew_ref):
    dA = jnp.clip(degA_ref[0, :, 0:1] + degA_ref[1, :, 0:1], 1.0)
    zA = (aggA_ref[0] + aggA_ref[1]) / dA
    uA = jnp.maximum(zA + b1A_ref[...], 0.0) @ w2A_ref[...] + b2A_ref[...]
    dB = jnp.clip(degB_ref[0, :, 0:1] + degB_ref[1, :, 0:1], 1.0)
    zB = (aggB_ref[0] + aggB_ref[1]) / dB
    uB = jnp.maximum(zB + b1B_ref[...], 0.0) @ w2B_ref[...] + b2B_ref[...]
    h2 = jnp.concatenate([uA, uB], axis=1)
    h2_ref[...] = h2
    hnew_ref[...] = (jnp.maximum(h2, 0.0) + hprev_ref[...]) * 0.5


def _post(aggA, degA, pA, aggB, degB, pB, hprev, p):
    n = hprev.shape[0]
    np_dst = aggA.shape[1]
    blk = 1264 if n > 1264 else np_dst
    grid = np_dst // blk
    return pl.pallas_call(
        _post_body,
        grid=(grid,),
        in_specs=[
            pl.BlockSpec((NUM_SC, blk, HID), lambda i: (0, i, 0)),
            pl.BlockSpec((NUM_SC, blk, 8), lambda i: (0, i, 0)),
            pl.BlockSpec((HID, HID), lambda i: (0, 0)),
            pl.BlockSpec((1, HID), lambda i: (0, 0)),
            pl.BlockSpec((1, HID), lambda i: (0, 0)),
            pl.BlockSpec((NUM_SC, blk, HID), lambda i: (0, i, 0)),
            pl.BlockSpec((NUM_SC, blk, 8), lambda i: (0, i, 0)),
            pl.BlockSpec((HID, HID), lambda i: (0, 0)),
            pl.BlockSpec((1, HID), lambda i: (0, 0)),
            pl.BlockSpec((1, HID), lambda i: (0, 0)),
            pl.BlockSpec((blk, EMB), lambda i: (i, 0)),
        ],
        out_specs=[pl.BlockSpec((blk, EMB), lambda i: (i, 0)),
                   pl.BlockSpec((blk, EMB), lambda i: (i, 0))],
        out_shape=[jax.ShapeDtypeStruct((n, EMB), jnp.float32),
                   jax.ShapeDtypeStruct((n, EMB), jnp.float32)],
    )(aggA, degA, p[pA + "_W2"], p[pA + "_b1"][None, :], p[pA + "_b2"][None, :],
      aggB, degB, p[pB + "_W2"], p[pB + "_b1"][None, :], p[pB + "_b2"][None, :],
      hprev)


def _pred_body(x_ref, w1_ref, b1_ref, w2_ref, b2_ref, o_ref):
    h = jnp.maximum(x_ref[...] @ w1_ref[...] + b1_ref[...], 0.0)
    o_ref[...] = h @ w2_ref[...] + b2_ref[...]


def _pred(x, p):
    out = pl.pallas_call(
        _pred_body,
        out_shape=jax.ShapeDtypeStruct((x.shape[0], 1), jnp.float32),
    )(x, p["pred_W1"], p["pred_b1"][None, :], p["pred_W2"],
      p["pred_b2"][None, :])
    return out[:, 0]


def _prep_edges(ei, ea, n_dst):
    src, dst, eaf = ei[0], ei[1], ea[:, 0]
    e = src.shape[0]
    e_pad = ((e + NW * 8 - 1) // (NW * 8)) * (NW * 8)
    if e_pad != e:
        pad = e_pad - e
        src = jnp.concatenate([src, jnp.zeros((pad,), jnp.int32)])
        dst = jnp.concatenate([dst, jnp.full((pad,), n_dst, jnp.int32)])
        eaf = jnp.concatenate([eaf, jnp.zeros((pad,), jnp.float32)])
    return src, dst, eaf


def kernel(x_cons, x_vals, x_obj, ea_c2v, ea_v2c, ea_v2o, ea_o2v, ea_c2o,
           ea_o2c, params, ei_c2v, ei_v2c, ei_v2o, ei_o2v, ei_c2o, ei_o2c):
    p = params
    ndst = {"c2v": N_V, "o2v": N_V, "v2c": N_C, "o2c": N_C,
            "v2o": N_O, "c2o": N_O}
    edges = {}
    for name, ei, ea in [("c2v", ei_c2v, ea_c2v), ("v2c", ei_v2c, ea_v2c),
                         ("v2o", ei_v2o, ea_v2o), ("o2v", ei_o2v, ea_o2v),
                         ("c2o", ei_c2o, ea_c2o), ("o2c", ei_o2c, ea_o2c)]:
        edges[name] = _prep_edges(ei, ea, ndst[name])

    h = {"cons": _encoder(x_cons, p, "enc_cons"),
         "vals": _encoder(x_vals, p, "enc_vals"),
         "obj": _encoder(x_obj, p, "enc_obj")}

    src_of = {"c2v": "cons", "c2o": "cons", "v2c": "vals", "v2o": "vals",
              "o2v": "obj", "o2c": "obj"}
    degs = {}
    last_vals = None
    for layer in range(2):
        y = {}
        y["c2v"], y["c2o"] = _pre(h["cons"], p["c2v_W1"], p["c2o_W1"])
        y["v2c"], y["v2o"] = _pre(h["vals"], p["v2c_W1"], p["v2o_W1"])
        y["o2v"], y["o2c"] = _pre(h["obj"], p["o2v_W1"], p["o2c_W1"])
        agg = {}
        for name in ["c2v", "v2c", "v2o", "o2v", "c2o", "o2c"]:
            src, dst, eaf = edges[name]
            outs = _spmm(y[name], src, dst, eaf, ndst[name],
                         with_deg=(layer == 0))
            agg[name] = outs[0]
            if layer == 0:
                degs[name] = outs[1]
        h2v, hv = _post(agg["c2v"], degs["c2v"], "c2v",
                        agg["o2v"], degs["o2v"], "o2v", h["vals"], p)
        h2c, hc = _post(agg["v2c"], degs["v2c"], "v2c",
                        agg["o2c"], degs["o2c"], "o2c", h["cons"], p)
        h2o, ho = _post(agg["v2o"], degs["v2o"], "v2o",
                        agg["c2o"], degs["c2o"], "c2o", h["obj"], p)
        h = {"vals": hv, "cons": hc, "obj": ho}
        last_vals = h2v

    return _pred(last_vals, p)


# speedup vs baseline: 10.4187x; 1.3298x over previous
import functools

import jax
import jax.numpy as jnp
from jax import lax
from jax.experimental import pallas as pl
from jax.experimental.pallas import tpu as pltpu
from jax.experimental.pallas import tpu_sc as plsc

N_C, N_V, N_O = 10000, 10000, 32
HID, EMB = 64, 128
NUM_SC, NUM_TILES = 2, 16
NW = NUM_SC * NUM_TILES

_MESH = plsc.VectorSubcoreMesh(
    core_axis_name="c", subcore_axis_name="s", num_cores=NUM_SC,
    num_subcores=NUM_TILES)


def _make_spmm(n_src, np_dst, e_pad, K, with_deg):
    EPT = e_pad // NW
    nchunk = EPT // K
    assert nchunk == 1 or nchunk % 2 == 0
    rpt = np_dst // NUM_TILES

    out_type = [jax.ShapeDtypeStruct((NUM_SC, np_dst, HID), jnp.float32)]
    if with_deg:
        out_type.append(jax.ShapeDtypeStruct((NUM_SC, np_dst, 8),
                                             jnp.float32))

    scratch = dict(
        ebuf0=pltpu.VMEM((2, K), jnp.int32),
        ebuf1=pltpu.VMEM((2, K), jnp.int32),
        eab0=pltpu.VMEM((K,), jnp.float32),
        eab1=pltpu.VMEM((K,), jnp.float32),
        rows0=pltpu.VMEM((K, HID), jnp.float32),
        rows1=pltpu.VMEM((K, HID), jnp.float32),
        sagg=pltpu.VMEM_SHARED((np_dst, HID), jnp.float32),
        gsem0=pltpu.SemaphoreType.DMA,
        gsem1=pltpu.SemaphoreType.DMA,
    )
    if with_deg:
        scratch["onesb"] = pltpu.VMEM((K, 8), jnp.float32)
        scratch["sdeg"] = pltpu.VMEM_SHARED((np_dst, 8), jnp.float32)

    def body(y_h, pack_h, ea_h, z64_h, z8_h, o8_h, *outs, ebuf0, ebuf1,
             eab0, eab1, rows0, rows1, sagg, gsem0, gsem1, onesb=None,
             sdeg=None):
        if with_deg:
            agg_h, deg_h = outs
        else:
            agg_h, = outs
        cid = lax.axis_index("c")
        sid = lax.axis_index("s")
        wid = cid * NUM_TILES + sid
        r0 = sid * rpt
        pltpu.sync_copy(z64_h.at[pl.ds(r0, rpt)], sagg.at[pl.ds(r0, rpt)])
        if with_deg:
            pltpu.sync_copy(z8_h.at[pl.ds(r0, rpt)], sdeg.at[pl.ds(r0, rpt)])
            pltpu.sync_copy(o8_h, onesb)
        plsc.subcore_barrier()

        base0 = wid * EPT

        def fetch(ci, ebuf, eab):
            b = pl.multiple_of(base0 + ci * K, 8)
            pltpu.sync_copy(pack_h.at[:, pl.ds(b, K)], ebuf)
            pltpu.sync_copy(ea_h.at[pl.ds(b, K)], eab)

        def gissue(ebuf, rows, gsem):
            return pltpu.async_copy(y_h.at[ebuf.at[0]], rows, gsem)

        def gwait(ebuf, rows, gsem):
            pltpu.make_async_copy(y_h.at[ebuf.at[0]], rows, gsem).wait()

        def scale_scatter(ebuf, eab, rows):
            def scale(g, c2):
                ea16 = eab[pl.ds(g * 16, 16)]
                for i in range(16):
                    w = ea16[i]
                    k = g * 16 + i
                    for j in range(HID // 16):
                        sl = (k, pl.ds(j * 16, 16))
                        rows[sl] = rows[sl] * w
                return c2

            lax.fori_loop(0, K // 16, scale, 0)
            pltpu.sync_copy(rows, sagg.at[ebuf.at[1]], add=True)
            if with_deg:
                pltpu.sync_copy(onesb, sdeg.at[ebuf.at[1]], add=True)

        if nchunk == 1:
            fetch(0, ebuf0, eab0)
            gissue(ebuf0, rows0, gsem0).wait()
            scale_scatter(ebuf0, eab0, rows0)
        else:
            fetch(0, ebuf0, eab0)
            gissue(ebuf0, rows0, gsem0)

            def pair(pi, carry):
                a = pi * 2
                gwait(ebuf0, rows0, gsem0)
                fetch(a + 1, ebuf1, eab1)
                gissue(ebuf1, rows1, gsem1)
                scale_scatter(ebuf0, eab0, rows0)
                gwait(ebuf1, rows1, gsem1)

                @pl.when(pi + 1 < nchunk // 2)
                def _():
                    fetch(a + 2, ebuf0, eab0)
                    gissue(ebuf0, rows0, gsem0)

                scale_scatter(ebuf1, eab1, rows1)
                return carry

            lax.fori_loop(0, nchunk // 2, pair, 0)

        plsc.subcore_barrier()
        pltpu.sync_copy(sagg.at[pl.ds(r0, rpt)], agg_h.at[cid, pl.ds(r0, rpt)])
        if with_deg:
            pltpu.sync_copy(sdeg.at[pl.ds(r0, rpt)],
                            deg_h.at[cid, pl.ds(r0, rpt)])

    return pl.kernel(
        body, out_type=out_type, mesh=_MESH, scratch_types=scratch,
        compiler_params=pltpu.CompilerParams(use_tc_tiling_on_sc=False))


@functools.lru_cache(maxsize=None)
def _spmm_fns(n_src, np_dst, e_pad, K, with_deg):
    return _make_spmm(n_src, np_dst, e_pad, K, with_deg)


def _spmm(y, pack, eaf, n_dst, with_deg):
    e_pad = pack.shape[1]
    ept = e_pad // NW
    K = 400 if ept % 400 == 0 else ept
    np_dst = ((n_dst + 127) // 128) * 128
    fn = _spmm_fns(y.shape[0], np_dst, e_pad, K, with_deg)
    z64 = jnp.zeros((np_dst, HID), jnp.float32)
    z8 = jnp.zeros((np_dst, 8), jnp.float32)
    o8 = jnp.ones((K, 8), jnp.float32)
    return fn(y, pack, eaf, z64, z8, o8)


def _enc_body(x_ref, w1_ref, b1_ref, g_ref, be_ref, w2_ref, b2_ref, o_ref):
    h = x_ref[...] @ w1_ref[...] + b1_ref[...]
    mu = jnp.mean(h, axis=0, keepdims=True)
    var = jnp.mean((h - mu) ** 2, axis=0, keepdims=True)
    h = (h - mu) / jnp.sqrt(var + 1e-5) * g_ref[...] + be_ref[...]
    o_ref[...] = jnp.maximum(h, 0.0) @ w2_ref[...] + b2_ref[...]


def _encoder(x, p, n):
    return pl.pallas_call(
        _enc_body,
        out_shape=jax.ShapeDtypeStruct((x.shape[0], EMB), jnp.float32),
    )(x, p[n + "_W1"], p[n + "_b1"][None, :], p[n + "_g"][None, :],
      p[n + "_be"][None, :], p[n + "_W2"], p[n + "_b2"][None, :])


def _pre_body(h_ref, wa_ref, wb_ref, oa_ref, ob_ref):
    h = h_ref[...]
    oa_ref[...] = h @ wa_ref[...]
    ob_ref[...] = h @ wb_ref[...]


def _pre(h, wa, wb):
    n = h.shape[0]
    return pl.pallas_call(
        _pre_body,
        out_shape=[jax.ShapeDtypeStruct((n, HID), jnp.float32),
                   jax.ShapeDtypeStruct((n, HID), jnp.float32)],
    )(h, wa, wb)


def _post_body(aggA_ref, degA_ref, w2A_ref, b1A_ref, b2A_ref,
               aggB_ref, degB_ref, w2B_ref, b1B_ref, b2B_ref,
               hprev_ref, h2_ref, hnew_ref):
    dA = jnp.clip(degA_ref[0, :, 0:1] + degA_ref[1, :, 0:1], 1.0)
    zA = (aggA_ref[0] + aggA_ref[1]) / dA
    uA = jnp.maximum(zA + b1A_ref[...], 0.0) @ w2A_ref[...] + b2A_ref[...]
    dB = jnp.clip(degB_ref[0, :, 0:1] + degB_ref[1, :, 0:1], 1.0)
    zB = (aggB_ref[0] + aggB_ref[1]) / dB
    uB = jnp.maximum(zB + b1B_ref[...], 0.0) @ w2B_ref[...] + b2B_ref[...]
    h2 = jnp.concatenate([uA, uB], axis=1)
    h2_ref[...] = h2
    hnew_ref[...] = (jnp.maximum(h2, 0.0) + hprev_ref[...]) * 0.5


def _post(aggA, degA, pA, aggB, degB, pB, hprev, p):
    n = hprev.shape[0]
    np_dst = aggA.shape[1]
    blk = 1264 if n > 1264 else np_dst
    grid = np_dst // blk
    return pl.pallas_call(
        _post_body,
        grid=(grid,),
        in_specs=[
            pl.BlockSpec((NUM_SC, blk, HID), lambda i: (0, i, 0)),
            pl.BlockSpec((NUM_SC, blk, 8), lambda i: (0, i, 0)),
            pl.BlockSpec((HID, HID), lambda i: (0, 0)),
            pl.BlockSpec((1, HID), lambda i: (0, 0)),
            pl.BlockSpec((1, HID), lambda i: (0, 0)),
            pl.BlockSpec((NUM_SC, blk, HID), lambda i: (0, i, 0)),
            pl.BlockSpec((NUM_SC, blk, 8), lambda i: (0, i, 0)),
            pl.BlockSpec((HID, HID), lambda i: (0, 0)),
            pl.BlockSpec((1, HID), lambda i: (0, 0)),
            pl.BlockSpec((1, HID), lambda i: (0, 0)),
            pl.BlockSpec((blk, EMB), lambda i: (i, 0)),
        ],
        out_specs=[pl.BlockSpec((blk, EMB), lambda i: (i, 0)),
                   pl.BlockSpec((blk, EMB), lambda i: (i, 0))],
        out_shape=[jax.ShapeDtypeStruct((n, EMB), jnp.float32),
                   jax.ShapeDtypeStruct((n, EMB), jnp.float32)],
    )(aggA, degA, p[pA + "_W2"], p[pA + "_b1"][None, :], p[pA + "_b2"][None, :],
      aggB, degB, p[pB + "_W2"], p[pB + "_b1"][None, :], p[pB + "_b2"][None, :],
      hprev)


def _pred_body(x_ref, w1_ref, b1_ref, w2_ref, b2_ref, o_ref):
    h = jnp.maximum(x_ref[...] @ w1_ref[...] + b1_ref[...], 0.0)
    o_ref[...] = h @ w2_ref[...] + b2_ref[...]


def _pred(x, p):
    out = pl.pallas_call(
        _pred_body,
        out_shape=jax.ShapeDtypeStruct((x.shape[0], 1), jnp.float32),
    )(x, p["pred_W1"], p["pred_b1"][None, :], p["pred_W2"],
      p["pred_b2"][None, :])
    return out[:, 0]


def _prep_edges(ei, ea, n_dst):
    src, dst, eaf = ei[0], ei[1], ea[:, 0]
    e = src.shape[0]
    e_pad = ((e + NW * 8 - 1) // (NW * 8)) * (NW * 8)
    if e_pad != e:
        pad = e_pad - e
        src = jnp.concatenate([src, jnp.zeros((pad,), jnp.int32)])
        dst = jnp.concatenate([dst, jnp.full((pad,), n_dst, jnp.int32)])
        eaf = jnp.concatenate([eaf, jnp.zeros((pad,), jnp.float32)])
    return jnp.stack([src, dst]), eaf


def kernel(x_cons, x_vals, x_obj, ea_c2v, ea_v2c, ea_v2o, ea_o2v, ea_c2o,
           ea_o2c, params, ei_c2v, ei_v2c, ei_v2o, ei_o2v, ei_c2o, ei_o2c):
    p = params
    ndst = {"c2v": N_V, "o2v": N_V, "v2c": N_C, "o2c": N_C,
            "v2o": N_O, "c2o": N_O}
    edges = {}
    for name, ei, ea in [("c2v", ei_c2v, ea_c2v), ("v2c", ei_v2c, ea_v2c),
                         ("v2o", ei_v2o, ea_v2o), ("o2v", ei_o2v, ea_o2v),
                         ("c2o", ei_c2o, ea_c2o), ("o2c", ei_o2c, ea_o2c)]:
        edges[name] = _prep_edges(ei, ea, ndst[name])

    h = {"cons": _encoder(x_cons, p, "enc_cons"),
         "vals": _encoder(x_vals, p, "enc_vals"),
         "obj": _encoder(x_obj, p, "enc_obj")}

    src_of = {"c2v": "cons", "c2o": "cons", "v2c": "vals", "v2o": "vals",
              "o2v": "obj", "o2c": "obj"}
    degs = {}
    last_vals = None
    for layer in range(2):
        y = {}
        y["c2v"], y["c2o"] = _pre(h["cons"], p["c2v_W1"], p["c2o_W1"])
        y["v2c"], y["v2o"] = _pre(h["vals"], p["v2c_W1"], p["v2o_W1"])
        y["o2v"], y["o2c"] = _pre(h["obj"], p["o2v_W1"], p["o2c_W1"])
        agg = {}
        for name in ["c2v", "v2c", "v2o", "o2v", "c2o", "o2c"]:
            pack, eaf = edges[name]
            outs = _spmm(y[name], pack, eaf, ndst[name],
                         with_deg=(layer == 0))
            agg[name] = outs[0]
            if layer == 0:
                degs[name] = outs[1]
        h2v, hv = _post(agg["c2v"], degs["c2v"], "c2v",
                        agg["o2v"], degs["o2v"], "o2v", h["vals"], p)
        h2c, hc = _post(agg["v2c"], degs["v2c"], "v2c",
                        agg["o2c"], degs["o2c"], "o2c", h["cons"], p)
        h2o, ho = _post(agg["v2o"], degs["v2o"], "v2o",
                        agg["c2o"], degs["c2o"], "c2o", h["obj"], p)
        h = {"vals": hv, "cons": hc, "obj": ho}
        last_vals = h2v

    return _pred(last_vals, p)


# speedup vs baseline: 11.8911x; 1.1413x over previous
import functools

import jax
import jax.numpy as jnp
from jax import lax
from jax.experimental import pallas as pl
from jax.experimental.pallas import tpu as pltpu
from jax.experimental.pallas import tpu_sc as plsc

N_C, N_V, N_O = 10000, 10000, 32
HID, EMB = 64, 128
NUM_SC, NUM_TILES = 2, 16
NW = NUM_SC * NUM_TILES

_MESH = plsc.VectorSubcoreMesh(
    core_axis_name="c", subcore_axis_name="s", num_cores=NUM_SC,
    num_subcores=NUM_TILES)


def _make_spmm(n_src, np_dst, e_pad, K, with_deg):
    EPT = e_pad // NW
    nchunk = EPT // K
    assert nchunk == 1 or nchunk % 2 == 0
    rpt = np_dst // NUM_TILES

    out_type = [jax.ShapeDtypeStruct((NUM_SC, np_dst, HID), jnp.float32)]
    if with_deg:
        out_type.append(jax.ShapeDtypeStruct((NUM_SC, np_dst, 8),
                                             jnp.float32))

    scratch = dict(
        ebuf0=pltpu.VMEM((2, K), jnp.int32),
        ebuf1=pltpu.VMEM((2, K), jnp.int32),
        eab0=pltpu.VMEM((K,), jnp.float32),
        eab1=pltpu.VMEM((K,), jnp.float32),
        rows0=pltpu.VMEM((K, HID), jnp.float32),
        rows1=pltpu.VMEM((K, HID), jnp.float32),
        sagg=pltpu.VMEM_SHARED((np_dst, HID), jnp.float32),
        gsem0=pltpu.SemaphoreType.DMA,
        gsem1=pltpu.SemaphoreType.DMA,
    )
    if with_deg:
        scratch["onesb"] = pltpu.VMEM((K, 8), jnp.float32)
        scratch["sdeg"] = pltpu.VMEM_SHARED((np_dst, 8), jnp.float32)

    def body(y_h, pack_h, ea_h, z64_h, z8_h, o8_h, *outs, ebuf0, ebuf1,
             eab0, eab1, rows0, rows1, sagg, gsem0, gsem1, onesb=None,
             sdeg=None):
        if with_deg:
            agg_h, deg_h = outs
        else:
            agg_h, = outs
        cid = lax.axis_index("c")
        sid = lax.axis_index("s")
        wid = cid * NUM_TILES + sid
        r0 = sid * rpt
        pltpu.sync_copy(z64_h.at[pl.ds(r0, rpt)], sagg.at[pl.ds(r0, rpt)])
        if with_deg:
            pltpu.sync_copy(z8_h.at[pl.ds(r0, rpt)], sdeg.at[pl.ds(r0, rpt)])
            pltpu.sync_copy(o8_h, onesb)
        plsc.subcore_barrier()

        base0 = wid * EPT

        def fetch(ci, ebuf, eab):
            b = pl.multiple_of(base0 + ci * K, 8)
            pltpu.sync_copy(pack_h.at[:, pl.ds(b, K)], ebuf)
            pltpu.sync_copy(ea_h.at[pl.ds(b, K)], eab)

        def gissue(ebuf, rows, gsem):
            return pltpu.async_copy(y_h.at[ebuf.at[0]], rows, gsem)

        def gwait(ebuf, rows, gsem):
            pltpu.make_async_copy(y_h.at[ebuf.at[0]], rows, gsem).wait()

        def scale_scatter(ebuf, eab, rows):
            @plsc.parallel_loop(0, K // 16, unroll=2)
            def _scale(g):
                ea16 = eab[pl.ds(g * 16, 16)]
                for i in range(16):
                    w = ea16[i]
                    k = g * 16 + i
                    for j in range(HID // 16):
                        sl = (k, pl.ds(j * 16, 16))
                        rows[sl] = rows[sl] * w
            pltpu.sync_copy(rows, sagg.at[ebuf.at[1]], add=True)
            if with_deg:
                pltpu.sync_copy(onesb, sdeg.at[ebuf.at[1]], add=True)

        if nchunk == 1:
            fetch(0, ebuf0, eab0)
            gissue(ebuf0, rows0, gsem0).wait()
            scale_scatter(ebuf0, eab0, rows0)
        else:
            fetch(0, ebuf0, eab0)
            gissue(ebuf0, rows0, gsem0)

            def pair(pi, carry):
                a = pi * 2
                gwait(ebuf0, rows0, gsem0)
                fetch(a + 1, ebuf1, eab1)
                gissue(ebuf1, rows1, gsem1)
                scale_scatter(ebuf0, eab0, rows0)
                gwait(ebuf1, rows1, gsem1)

                @pl.when(pi + 1 < nchunk // 2)
                def _():
                    fetch(a + 2, ebuf0, eab0)
                    gissue(ebuf0, rows0, gsem0)

                scale_scatter(ebuf1, eab1, rows1)
                return carry

            lax.fori_loop(0, nchunk // 2, pair, 0)

        plsc.subcore_barrier()
        pltpu.sync_copy(sagg.at[pl.ds(r0, rpt)], agg_h.at[cid, pl.ds(r0, rpt)])
        if with_deg:
            pltpu.sync_copy(sdeg.at[pl.ds(r0, rpt)],
                            deg_h.at[cid, pl.ds(r0, rpt)])

    return pl.kernel(
        body, out_type=out_type, mesh=_MESH, scratch_types=scratch,
        compiler_params=pltpu.CompilerParams(use_tc_tiling_on_sc=False))


@functools.lru_cache(maxsize=None)
def _spmm_fns(n_src, np_dst, e_pad, K, with_deg):
    return _make_spmm(n_src, np_dst, e_pad, K, with_deg)


def _spmm(y, pack, eaf, n_dst, with_deg):
    e_pad = pack.shape[1]
    ept = e_pad // NW
    K = 400 if ept % 400 == 0 else ept
    np_dst = ((n_dst + 127) // 128) * 128
    fn = _spmm_fns(y.shape[0], np_dst, e_pad, K, with_deg)
    z64 = jnp.zeros((np_dst, HID), jnp.float32)
    z8 = jnp.zeros((np_dst, 8), jnp.float32)
    o8 = jnp.ones((K, 8), jnp.float32)
    return fn(y, pack, eaf, z64, z8, o8)


def _enc_body(x_ref, w1_ref, b1_ref, g_ref, be_ref, w2_ref, b2_ref, o_ref):
    h = x_ref[...] @ w1_ref[...] + b1_ref[...]
    mu = jnp.mean(h, axis=0, keepdims=True)
    var = jnp.mean((h - mu) ** 2, axis=0, keepdims=True)
    h = (h - mu) / jnp.sqrt(var + 1e-5) * g_ref[...] + be_ref[...]
    o_ref[...] = jnp.maximum(h, 0.0) @ w2_ref[...] + b2_ref[...]


def _encoder(x, p, n):
    return pl.pallas_call(
        _enc_body,
        out_shape=jax.ShapeDtypeStruct((x.shape[0], EMB), jnp.float32),
    )(x, p[n + "_W1"], p[n + "_b1"][None, :], p[n + "_g"][None, :],
      p[n + "_be"][None, :], p[n + "_W2"], p[n + "_b2"][None, :])


def _pre_body(h_ref, wa_ref, wb_ref, oa_ref, ob_ref):
    h = h_ref[...]
    oa_ref[...] = h @ wa_ref[...]
    ob_ref[...] = h @ wb_ref[...]


def _pre(h, wa, wb):
    n = h.shape[0]
    return pl.pallas_call(
        _pre_body,
        out_shape=[jax.ShapeDtypeStruct((n, HID), jnp.float32),
                   jax.ShapeDtypeStruct((n, HID), jnp.float32)],
    )(h, wa, wb)


def _post_body(aggA_ref, degA_ref, w2A_ref, b1A_ref, b2A_ref,
               aggB_ref, degB_ref, w2B_ref, b1B_ref, b2B_ref,
               hprev_ref, h2_ref, hnew_ref):
    dA = jnp.clip(degA_ref[0, :, 0:1] + degA_ref[1, :, 0:1], 1.0)
    zA = (aggA_ref[0] + aggA_ref[1]) / dA
    uA = jnp.maximum(zA + b1A_ref[...], 0.0) @ w2A_ref[...] + b2A_ref[...]
    dB = jnp.clip(degB_ref[0, :, 0:1] + degB_ref[1, :, 0:1], 1.0)
    zB = (aggB_ref[0] + aggB_ref[1]) / dB
    uB = jnp.maximum(zB + b1B_ref[...], 0.0) @ w2B_ref[...] + b2B_ref[...]
    h2 = jnp.concatenate([uA, uB], axis=1)
    h2_ref[...] = h2
    hnew_ref[...] = (jnp.maximum(h2, 0.0) + hprev_ref[...]) * 0.5


def _post(aggA, degA, pA, aggB, degB, pB, hprev, p):
    n = hprev.shape[0]
    np_dst = aggA.shape[1]
    blk = 1264 if n > 1264 else np_dst
    grid = np_dst // blk
    return pl.pallas_call(
        _post_body,
        grid=(grid,),
        in_specs=[
            pl.BlockSpec((NUM_SC, blk, HID), lambda i: (0, i, 0)),
            pl.BlockSpec((NUM_SC, blk, 8), lambda i: (0, i, 0)),
            pl.BlockSpec((HID, HID), lambda i: (0, 0)),
            pl.BlockSpec((1, HID), lambda i: (0, 0)),
            pl.BlockSpec((1, HID), lambda i: (0, 0)),
            pl.BlockSpec((NUM_SC, blk, HID), lambda i: (0, i, 0)),
            pl.BlockSpec((NUM_SC, blk, 8), lambda i: (0, i, 0)),
            pl.BlockSpec((HID, HID), lambda i: (0, 0)),
            pl.BlockSpec((1, HID), lambda i: (0, 0)),
            pl.BlockSpec((1, HID), lambda i: (0, 0)),
            pl.BlockSpec((blk, EMB), lambda i: (i, 0)),
        ],
        out_specs=[pl.BlockSpec((blk, EMB), lambda i: (i, 0)),
                   pl.BlockSpec((blk, EMB), lambda i: (i, 0))],
        out_shape=[jax.ShapeDtypeStruct((n, EMB), jnp.float32),
                   jax.ShapeDtypeStruct((n, EMB), jnp.float32)],
    )(aggA, degA, p[pA + "_W2"], p[pA + "_b1"][None, :], p[pA + "_b2"][None, :],
      aggB, degB, p[pB + "_W2"], p[pB + "_b1"][None, :], p[pB + "_b2"][None, :],
      hprev)


def _pred_body(x_ref, w1_ref, b1_ref, w2_ref, b2_ref, o_ref):
    h = jnp.maximum(x_ref[...] @ w1_ref[...] + b1_ref[...], 0.0)
    o_ref[...] = h @ w2_ref[...] + b2_ref[...]


def _pred(x, p):
    out = pl.pallas_call(
        _pred_body,
        out_shape=jax.ShapeDtypeStruct((x.shape[0], 1), jnp.float32),
    )(x, p["pred_W1"], p["pred_b1"][None, :], p["pred_W2"],
      p["pred_b2"][None, :])
    return out[:, 0]


def _prep_edges(ei, ea, n_dst):
    src, dst, eaf = ei[0], ei[1], ea[:, 0]
    e = src.shape[0]
    e_pad = ((e + NW * 8 - 1) // (NW * 8)) * (NW * 8)
    if e_pad != e:
        pad = e_pad - e
        src = jnp.concatenate([src, jnp.zeros((pad,), jnp.int32)])
        dst = jnp.concatenate([dst, jnp.full((pad,), n_dst, jnp.int32)])
        eaf = jnp.concatenate([eaf, jnp.zeros((pad,), jnp.float32)])
    return jnp.stack([src, dst]), eaf


def kernel(x_cons, x_vals, x_obj, ea_c2v, ea_v2c, ea_v2o, ea_o2v, ea_c2o,
           ea_o2c, params, ei_c2v, ei_v2c, ei_v2o, ei_o2v, ei_c2o, ei_o2c):
    p = params
    ndst = {"c2v": N_V, "o2v": N_V, "v2c": N_C, "o2c": N_C,
            "v2o": N_O, "c2o": N_O}
    edges = {}
    for name, ei, ea in [("c2v", ei_c2v, ea_c2v), ("v2c", ei_v2c, ea_v2c),
                         ("v2o", ei_v2o, ea_v2o), ("o2v", ei_o2v, ea_o2v),
                         ("c2o", ei_c2o, ea_c2o), ("o2c", ei_o2c, ea_o2c)]:
        edges[name] = _prep_edges(ei, ea, ndst[name])

    h = {"cons": _encoder(x_cons, p, "enc_cons"),
         "vals": _encoder(x_vals, p, "enc_vals"),
         "obj": _encoder(x_obj, p, "enc_obj")}

    src_of = {"c2v": "cons", "c2o": "cons", "v2c": "vals", "v2o": "vals",
              "o2v": "obj", "o2c": "obj"}
    degs = {}
    last_vals = None
    for layer in range(2):
        y = {}
        y["c2v"], y["c2o"] = _pre(h["cons"], p["c2v_W1"], p["c2o_W1"])
        y["v2c"], y["v2o"] = _pre(h["vals"], p["v2c_W1"], p["v2o_W1"])
        y["o2v"], y["o2c"] = _pre(h["obj"], p["o2v_W1"], p["o2c_W1"])
        agg = {}
        for name in ["c2v", "v2c", "v2o", "o2v", "c2o", "o2c"]:
            pack, eaf = edges[name]
            outs = _spmm(y[name], pack, eaf, ndst[name],
                         with_deg=(layer == 0))
            agg[name] = outs[0]
            if layer == 0:
                degs[name] = outs[1]
        h2v, hv = _post(agg["c2v"], degs["c2v"], "c2v",
                        agg["o2v"], degs["o2v"], "o2v", h["vals"], p)
        h2c, hc = _post(agg["v2c"], degs["v2c"], "v2c",
                        agg["o2c"], degs["o2c"], "o2c", h["cons"], p)
        h2o, ho = _post(agg["v2o"], degs["v2o"], "v2o",
                        agg["c2o"], degs["c2o"], "c2o", h["obj"], p)
        h = {"vals": hv, "cons": hc, "obj": ho}
        last_vals = h2v

    return _pred(last_vals, p)
